# Initial kernel scaffold; baseline (speedup 1.0000x reference)
import jax
import jax.numpy as jnp
from jax.experimental import pallas as pl


def kernel(state, c1_w, c1_b, c2_w, c2_b, c3_w, c3_b, q1_w, q1_b, q2_w, q2_b, i1_w, i1_b, i2_w, i2_b):
    raise NotImplementedError("write your pallas kernel here")


# speedup vs baseline: 26.0008x; 26.0008x over previous
import jax
import jax.numpy as jnp
from jax.experimental import pallas as pl
from jax.experimental.pallas import tpu as pltpu


def _cp():
    return pltpu.CompilerParams(
        dimension_semantics=("parallel",),
        vmem_limit_bytes=64 * 1024 * 1024,
    )


def _pick_tile(b: int, want: int) -> int:
    t = want
    while b % t:
        t //= 2
    return t


_S1 = (0, 1, 21, 22)
_S2 = (0, 1, 10, 11)
_S3 = (0, 1, 2, 10, 11, 12, 20, 21, 22)

_ROWS1 = 448
_ROWS2 = 128


def _conv1(x1, w1, b1, tb):
    rows = x1.shape[0]
    blk = tb * _ROWS1
    r = blk - 24

    def body(x_ref, w_ref, b_ref, o_ref):
        acc = None
        for t, s in enumerate(_S1):
            d = jnp.dot(x_ref[s:s + r, :], w_ref[64 * t:64 * (t + 1), :],
                        preferred_element_type=jnp.float32)
            acc = d if acc is None else acc + d
        o_ref[0:r, :] = jnp.maximum(acc + b_ref[...], 0.0).astype(o_ref.dtype)

    return pl.pallas_call(
        body,
        out_shape=jax.ShapeDtypeStruct((rows, 32), jnp.bfloat16),
        grid=(rows // blk,),
        in_specs=[
            pl.BlockSpec((blk, 64), lambda i: (i, 0)),
            pl.BlockSpec((256, 32), lambda i: (0, 0)),
            pl.BlockSpec((1, 32), lambda i: (0, 0)),
        ],
        out_specs=pl.BlockSpec((blk, 32), lambda i: (i, 0)),
        compiler_params=_cp(),
    )(x1, w1, b1)


def _conv23(x2, w2, b2, w3, b3, tb):
    rows = x2.shape[0]
    blk = tb * _ROWS2
    r2 = blk - 16
    r3 = blk - 40

    def body(x_ref, w2_ref, b2_ref, w3_ref, b3_ref, o_ref):
        acc = None
        for t, s in enumerate(_S2):
            d = jnp.dot(x_ref[s:s + r2, :], w2_ref[128 * t:128 * (t + 1), :],
                        preferred_element_type=jnp.float32)
            acc = d if acc is None else acc + d
        g = jnp.maximum(acc + b2_ref[...], 0.0).astype(jnp.bfloat16)
        acc3 = None
        for t, s in enumerate(_S3):
            d = jnp.dot(g[s:s + r3, :], w3_ref[64 * t:64 * (t + 1), :],
                        preferred_element_type=jnp.float32)
            acc3 = d if acc3 is None else acc3 + d
        o_ref[0:r3, :] = jnp.maximum(acc3 + b3_ref[...], 0.0).astype(o_ref.dtype)

    return pl.pallas_call(
        body,
        out_shape=jax.ShapeDtypeStruct((rows, 64), jnp.bfloat16),
        grid=(rows // blk,),
        in_specs=[
            pl.BlockSpec((blk, 128), lambda i: (i, 0)),
            pl.BlockSpec((512, 64), lambda i: (0, 0)),
            pl.BlockSpec((1, 64), lambda i: (0, 0)),
            pl.BlockSpec((576, 64), lambda i: (0, 0)),
            pl.BlockSpec((1, 64), lambda i: (0, 0)),
        ],
        out_specs=pl.BlockSpec((blk, 64), lambda i: (i, 0)),
        compiler_params=_cp(),
    )(x2, w2, b2, w3, b3)


def _heads(feats, wh, bh, w2, b2, tb, a):
    b = feats.shape[0]

    def body(f_ref, wh_ref, bh_ref, w2_ref, b2_ref, q_ref, lsm_ref, i_ref):
        h = jnp.dot(f_ref[...], wh_ref[...], preferred_element_type=jnp.float32)
        h = jnp.maximum(h + bh_ref[...], 0.0).astype(jnp.bfloat16)
        o = jnp.dot(h, w2_ref[...], preferred_element_type=jnp.float32)
        o = o + b2_ref[...]
        q_ref[...] = o[:, 0:128]
        iv = o[:, 128:256]
        i_ref[...] = iv
        col = jax.lax.broadcasted_iota(jnp.int32, iv.shape, 1)
        valid = col < a
        m = jnp.max(jnp.where(valid, iv, -jnp.inf), axis=-1, keepdims=True)
        s = iv - m
        e = jnp.where(valid, jnp.exp(s), 0.0)
        lsm_ref[...] = s - jnp.log(jnp.sum(e, axis=-1, keepdims=True))

    outs = pl.pallas_call(
        body,
        out_shape=(
            jax.ShapeDtypeStruct((b, 128), jnp.float32),
            jax.ShapeDtypeStruct((b, 128), jnp.float32),
            jax.ShapeDtypeStruct((b, 128), jnp.float32),
        ),
        grid=(b // tb,),
        in_specs=[
            pl.BlockSpec((tb, 3136), lambda i: (i, 0)),
            pl.BlockSpec((3136, 1024), lambda i: (0, 0)),
            pl.BlockSpec((1, 1024), lambda i: (0, 0)),
            pl.BlockSpec((1024, 256), lambda i: (0, 0)),
            pl.BlockSpec((1, 256), lambda i: (0, 0)),
        ],
        out_specs=(
            pl.BlockSpec((tb, 128), lambda i: (i, 0)),
            pl.BlockSpec((tb, 128), lambda i: (i, 0)),
            pl.BlockSpec((tb, 128), lambda i: (i, 0)),
        ),
        compiler_params=_cp(),
    )(feats, wh, bh, w2, b2)
    return outs


def kernel(state, c1_w, c1_b, c2_w, c2_b, c3_w, c3_b,
           q1_w, q1_b, q2_w, q2_b, i1_w, i1_b, i2_w, i2_b):
    B = state.shape[0]
    A = q2_w.shape[1]
    bf = jnp.bfloat16

    xb = state.astype(bf).reshape(B, 4, 21, 4, 21, 4)
    xb = xb.transpose(0, 2, 4, 3, 5, 1).reshape(B, 441, 64)
    x1 = jnp.pad(xb, ((0, 0), (0, _ROWS1 - 441), (0, 0))).reshape(B * _ROWS1, 64)
    w1 = c1_w.reshape(2, 4, 2, 4, 4, 32).transpose(0, 2, 1, 3, 4, 5)
    w1 = w1.reshape(256, 32).astype(bf)

    tb1 = _pick_tile(B, 16)
    y1 = _conv1(x1, w1, c1_b, tb1)

    y1 = y1.reshape(B, _ROWS1, 32)[:, :441].reshape(B, 21, 21, 32)[:, :20, :20]
    y1 = y1.reshape(B, 10, 2, 10, 2, 32).transpose(0, 1, 3, 2, 4, 5)
    y1 = y1.reshape(B, 100, 128)
    x2 = jnp.pad(y1, ((0, 0), (0, _ROWS2 - 100), (0, 0))).reshape(B * _ROWS2, 128)
    w2 = c2_w.reshape(2, 2, 2, 2, 32, 64).transpose(0, 2, 1, 3, 4, 5)
    w2 = w2.reshape(512, 64).astype(bf)
    w3 = c3_w.reshape(576, 64).astype(bf)

    tb2 = _pick_tile(B, 16)
    z = _conv23(x2, w2, c2_b, w3, c3_b, tb2)

    z = z.reshape(B, _ROWS2, 64)[:, :100].reshape(B, 10, 10, 64)[:, :7, :7]
    feats = z.transpose(0, 3, 1, 2).reshape(B, 3136)

    wh = jnp.concatenate([q1_w, i1_w], axis=1).astype(bf)
    bh = jnp.concatenate([q1_b, i1_b], axis=1)
    pad_a = ((0, 0), (0, 128 - A))
    z512 = jnp.zeros((512, 128), jnp.float32)
    w2h = jnp.concatenate([
        jnp.concatenate([jnp.pad(q2_w, pad_a), z512], axis=1),
        jnp.concatenate([z512, jnp.pad(i2_w, pad_a)], axis=1),
    ], axis=0).astype(bf)
    b2h = jnp.concatenate([jnp.pad(q2_b, pad_a), jnp.pad(i2_b, pad_a)], axis=1)

    tbh = _pick_tile(B, 128)
    q, lsm, i_out = _heads(feats, wh, bh, w2h, b2h, tbh, A)
    return q[:, :A], lsm[:, :A], i_out[:, :A]
